# Initial kernel scaffold; baseline (speedup 1.0000x reference)
import functools

import jax
import jax.numpy as jnp
from jax import lax
from jax.experimental import pallas as pl
from jax.experimental.pallas import tpu as pltpu
from jax.experimental.pallas import tpu_sc as plsc

_CHUNK = 128


@functools.cache
def _make_gather(n_rows, d):
    info = plsc.get_sparse_core_info()
    nc, ns = info.num_cores, info.num_subcores
    nw = nc * ns
    per_w = n_rows // nw
    n_chunks = per_w // _CHUNK
    mesh = plsc.VectorSubcoreMesh(core_axis_name="c", subcore_axis_name="s")

    @functools.partial(
        pl.kernel,
        mesh=mesh,
        out_type=jax.ShapeDtypeStruct((n_rows, d), jnp.float32),
        scratch_types=[
            pltpu.VMEM((n_chunks, _CHUNK), jnp.int32),
            pltpu.VMEM((_CHUNK, d), jnp.float32),
            pltpu.SemaphoreType.DMA,
        ],
    )
    def gather_kernel(table_hbm, idx_hbm, out_hbm, idx_v, rows_v, sem):
        wid = lax.axis_index("s") * nc + lax.axis_index("c")
        pltpu.sync_copy(idx_hbm.at[pl.ds(wid * n_chunks, n_chunks)], idx_v)

        def body(j, _):
            pltpu.async_copy(table_hbm.at[idx_v.at[j]], rows_v, sem).wait()
            pltpu.sync_copy(
                rows_v, out_hbm.at[pl.ds(wid * per_w + j * _CHUNK, _CHUNK)]
            )
            return 0

        lax.fori_loop(0, n_chunks, body, 0)

    return gather_kernel


def kernel(pe, timesteps, index_select):
    if timesteps.ndim == 1:
        return pe[: timesteps.shape[0]]
    b, l = timesteps.shape
    n = b * l
    d = pe.shape[1]
    idx = (timesteps.reshape(-1) * index_select).astype(jnp.int32)
    idx2d = idx.reshape(n // _CHUNK, _CHUNK)
    out = _make_gather(n, d)(pe, idx2d)
    return out.reshape(b, l, d)


# speedup vs baseline: 5.5298x; 5.5298x over previous
import functools

import jax
import jax.numpy as jnp
from jax import lax
from jax.experimental import pallas as pl
from jax.experimental.pallas import tpu as pltpu
from jax.experimental.pallas import tpu_sc as plsc

_CHUNK = 128


@functools.cache
def _make_gather(n_rows, d):
    info = plsc.get_sparse_core_info()
    nc, ns = info.num_cores, info.num_subcores
    nw = nc * ns
    per_w = n_rows // nw
    n_chunks = per_w // _CHUNK
    mesh = plsc.VectorSubcoreMesh(core_axis_name="c", subcore_axis_name="s")

    @functools.partial(
        pl.kernel,
        mesh=mesh,
        out_type=jax.ShapeDtypeStruct((n_rows, d), jnp.float32),
        scratch_types=[
            pltpu.VMEM((n_chunks, _CHUNK), jnp.int32),
            pltpu.VMEM((_CHUNK, d), jnp.float32),
            pltpu.SemaphoreType.DMA,
        ],
    )
    def gather_kernel(table_hbm, idx_hbm, out_hbm, idx_v, rows_v, sem):
        wid = lax.axis_index("s") * nc + lax.axis_index("c")
        pltpu.sync_copy(idx_hbm.at[wid], idx_v)

        def body(j, _):
            pltpu.async_copy(table_hbm.at[idx_v.at[j]], rows_v, sem).wait()
            pltpu.sync_copy(
                rows_v, out_hbm.at[pl.ds(wid * per_w + j * _CHUNK, _CHUNK)]
            )
            return 0

        lax.fori_loop(0, n_chunks, body, 0)

    return gather_kernel


def kernel(pe, timesteps, index_select):
    if timesteps.ndim == 1:
        return pe[: timesteps.shape[0]]
    b, l = timesteps.shape
    n = b * l
    d = pe.shape[1]
    idx = (timesteps.reshape(-1) * index_select).astype(jnp.int32)
    info = plsc.get_sparse_core_info()
    nw = info.num_cores * info.num_subcores
    idx3d = idx.reshape(nw, n // (nw * _CHUNK), _CHUNK)
    out = _make_gather(n, d)(pe, idx3d)
    return out.reshape(b, l, d)


# speedup vs baseline: 7.5183x; 1.3596x over previous
import functools

import jax
import jax.numpy as jnp
from jax import lax
from jax.experimental import pallas as pl
from jax.experimental.pallas import tpu as pltpu
from jax.experimental.pallas import tpu_sc as plsc

_CHUNK = 128
_GPS = 2
_SUPER = _CHUNK * _GPS


@functools.cache
def _make_gather(n_rows, d):
    info = plsc.get_sparse_core_info()
    nc, ns = info.num_cores, info.num_subcores
    nw = nc * ns
    per_w = n_rows // nw
    n_chunks = per_w // _CHUNK
    n_super = per_w // _SUPER
    mesh = plsc.VectorSubcoreMesh(core_axis_name="c", subcore_axis_name="s")

    @functools.partial(
        pl.kernel,
        mesh=mesh,
        out_type=jax.ShapeDtypeStruct((n_rows, d), jnp.float32),
        scratch_types=[
            pltpu.VMEM((n_chunks, _CHUNK), jnp.int32),
            pltpu.VMEM((2, _SUPER, d), jnp.float32),
            pltpu.SemaphoreType.DMA((2,)),
            pltpu.SemaphoreType.DMA((2,)),
        ],
    )
    def gather_kernel(table_hbm, idx_hbm, out_hbm, idx_v, rows_v, gsem, wsem):
        wid = lax.axis_index("s") * nc + lax.axis_index("c")
        base = wid * per_w
        pltpu.sync_copy(idx_hbm.at[wid], idx_v)

        def fire_gathers(u, slot):
            for g in range(_GPS):
                pltpu.async_copy(
                    table_hbm.at[idx_v.at[u * _GPS + g]],
                    rows_v.at[slot, pl.ds(g * _CHUNK, _CHUNK)],
                    gsem.at[slot],
                )

        def wait_gathers(slot):
            pltpu.make_async_copy(
                out_hbm.at[pl.ds(0, _SUPER)], rows_v.at[slot], gsem.at[slot]
            ).wait()

        def write_desc(u, slot):
            return pltpu.make_async_copy(
                rows_v.at[slot],
                out_hbm.at[pl.ds(base + u * _SUPER, _SUPER)],
                wsem.at[slot],
            )

        fire_gathers(0, 0)

        def body(u, _):
            slot = lax.rem(u, 2)
            nslot = 1 - slot

            @pl.when(u >= 1)
            def _():
                write_desc(u - 1, nslot).wait()

            @pl.when(u + 1 < n_super)
            def _():
                fire_gathers(u + 1, nslot)

            wait_gathers(slot)
            write_desc(u, slot).start()
            return 0

        lax.fori_loop(0, n_super, body, 0)
        write_desc(n_super - 1, lax.rem(n_super - 1, 2)).wait()

    return gather_kernel


def kernel(pe, timesteps, index_select):
    if timesteps.ndim == 1:
        return pe[: timesteps.shape[0]]
    b, l = timesteps.shape
    n = b * l
    d = pe.shape[1]
    idx = (timesteps.reshape(-1) * index_select).astype(jnp.int32)
    info = plsc.get_sparse_core_info()
    nw = info.num_cores * info.num_subcores
    idx3d = idx.reshape(nw, n // (nw * _CHUNK), _CHUNK)
    out = _make_gather(n, d)(pe, idx3d)
    return out.reshape(b, l, d)


# speedup vs baseline: 7.5602x; 1.0056x over previous
import functools

import jax
import jax.numpy as jnp
from jax import lax
from jax.experimental import pallas as pl
from jax.experimental.pallas import tpu as pltpu
from jax.experimental.pallas import tpu_sc as plsc

_CHUNK = 128
_GPS = 2
_SUPER = _CHUNK * _GPS


@functools.cache
def _make_gather(n_rows, d):
    info = plsc.get_sparse_core_info()
    nc, ns = info.num_cores, info.num_subcores
    nw = nc * ns
    per_w = n_rows // nw
    n_chunks = per_w // _CHUNK
    n_super = per_w // _SUPER
    mesh = plsc.VectorSubcoreMesh(core_axis_name="c", subcore_axis_name="s")

    @functools.partial(
        pl.kernel,
        mesh=mesh,
        out_type=jax.ShapeDtypeStruct((n_rows, d), jnp.float32),
        scratch_types=[
            pltpu.VMEM((n_chunks, _CHUNK), jnp.int32),
            pltpu.VMEM((3, _SUPER, d), jnp.float32),
            pltpu.SemaphoreType.DMA((3,)),
            pltpu.SemaphoreType.DMA((3,)),
        ],
    )
    def gather_kernel(table_hbm, idx_hbm, out_hbm, idx_v, rows_v, gsem, wsem):
        wid = lax.axis_index("s") * nc + lax.axis_index("c")
        base = wid * per_w
        pltpu.sync_copy(idx_hbm.at[wid], idx_v)

        def fire_gathers(u, slot):
            for g in range(_GPS):
                pltpu.async_copy(
                    table_hbm.at[idx_v.at[u * _GPS + g]],
                    rows_v.at[slot, pl.ds(g * _CHUNK, _CHUNK)],
                    gsem.at[slot],
                )

        def wait_gathers(slot):
            pltpu.make_async_copy(
                out_hbm.at[pl.ds(0, _SUPER)], rows_v.at[slot], gsem.at[slot]
            ).wait()

        def write_desc(u, slot):
            return pltpu.make_async_copy(
                rows_v.at[slot],
                out_hbm.at[pl.ds(base + u * _SUPER, _SUPER)],
                wsem.at[slot],
            )

        fire_gathers(0, 0)
        fire_gathers(1, 1)

        def body(u, _):
            slot = lax.rem(u, 3)

            @pl.when(u >= 1)
            def _():
                write_desc(u - 1, lax.rem(u - 1, 3)).wait()

            @pl.when(u + 2 < n_super)
            def _():
                fire_gathers(u + 2, lax.rem(u + 2, 3))

            wait_gathers(slot)
            write_desc(u, slot).start()
            return 0

        lax.fori_loop(0, n_super, body, 0)
        write_desc(n_super - 1, lax.rem(n_super - 1, 3)).wait()

    return gather_kernel


def kernel(pe, timesteps, index_select):
    if timesteps.ndim == 1:
        return pe[: timesteps.shape[0]]
    b, l = timesteps.shape
    n = b * l
    d = pe.shape[1]
    idx = (timesteps.reshape(-1) * index_select).astype(jnp.int32)
    info = plsc.get_sparse_core_info()
    nw = info.num_cores * info.num_subcores
    idx3d = idx.reshape(nw, n // (nw * _CHUNK), _CHUNK)
    out = _make_gather(n, d)(pe, idx3d)
    return out.reshape(b, l, d)


# speedup vs baseline: 10.9937x; 1.4542x over previous
import functools

import jax
import jax.numpy as jnp
from jax import lax
from jax.experimental import pallas as pl
from jax.experimental.pallas import tpu as pltpu
from jax.experimental.pallas import tpu_sc as plsc

_CHUNK = 128


@functools.cache
def _make_gather(n_rows, d, n_table):
    info = plsc.get_sparse_core_info()
    nc, ns = info.num_cores, info.num_subcores
    nw = nc * ns
    per_w = n_rows // nw
    n_chunks = per_w // _CHUNK
    t_per_s = n_table // ns
    mesh = plsc.VectorSubcoreMesh(core_axis_name="c", subcore_axis_name="s")

    @functools.partial(
        pl.kernel,
        mesh=mesh,
        out_type=jax.ShapeDtypeStruct((n_rows, d), jnp.float32),
        scratch_types=[
            pltpu.VMEM((n_chunks, _CHUNK), jnp.int32),
            pltpu.VMEM((2, _CHUNK, d), jnp.float32),
            pltpu.VMEM_SHARED((n_table, d), jnp.float32),
            pltpu.SemaphoreType.DMA((2,)),
            pltpu.SemaphoreType.DMA((2,)),
        ],
    )
    def gather_kernel(table_hbm, idx_hbm, out_hbm, idx_v, rows_v, tab_sh,
                      gsem, wsem):
        sid = lax.axis_index("s")
        wid = sid * nc + lax.axis_index("c")
        base = wid * per_w
        pltpu.sync_copy(
            table_hbm.at[pl.ds(sid * t_per_s, t_per_s)],
            tab_sh.at[pl.ds(sid * t_per_s, t_per_s)],
        )
        pltpu.sync_copy(idx_hbm.at[wid], idx_v)
        plsc.subcore_barrier()

        def fire_gather(u, slot):
            pltpu.async_copy(
                tab_sh.at[idx_v.at[u]], rows_v.at[slot], gsem.at[slot]
            )

        def wait_gather(slot):
            pltpu.make_async_copy(
                out_hbm.at[pl.ds(0, _CHUNK)], rows_v.at[slot], gsem.at[slot]
            ).wait()

        def write_desc(u, slot):
            return pltpu.make_async_copy(
                rows_v.at[slot],
                out_hbm.at[pl.ds(base + u * _CHUNK, _CHUNK)],
                wsem.at[slot],
            )

        fire_gather(0, 0)

        def body(u, _):
            slot = lax.rem(u, 2)
            nslot = 1 - slot

            @pl.when(u >= 1)
            def _():
                write_desc(u - 1, nslot).wait()

            @pl.when(u + 1 < n_chunks)
            def _():
                fire_gather(u + 1, nslot)

            wait_gather(slot)
            write_desc(u, slot).start()
            return 0

        lax.fori_loop(0, n_chunks, body, 0)
        write_desc(n_chunks - 1, lax.rem(n_chunks - 1, 2)).wait()

    return gather_kernel


def kernel(pe, timesteps, index_select):
    if timesteps.ndim == 1:
        return pe[: timesteps.shape[0]]
    b, l = timesteps.shape
    n = b * l
    d = pe.shape[1]
    idx = (timesteps.reshape(-1) * index_select).astype(jnp.int32)
    info = plsc.get_sparse_core_info()
    nw = info.num_cores * info.num_subcores
    idx3d = idx.reshape(nw, n // (nw * _CHUNK), _CHUNK)
    out = _make_gather(n, d, pe.shape[0])(pe, idx3d)
    return out.reshape(b, l, d)


# speedup vs baseline: 11.2178x; 1.0204x over previous
import functools

import jax
import jax.numpy as jnp
from jax import lax
from jax.experimental import pallas as pl
from jax.experimental.pallas import tpu as pltpu
from jax.experimental.pallas import tpu_sc as plsc

_CHUNK = 128


@functools.cache
def _make_gather(n_rows, d, n_table):
    info = plsc.get_sparse_core_info()
    nc, ns = info.num_cores, info.num_subcores
    nw = nc * ns
    per_w = n_rows // nw
    n_chunks = per_w // _CHUNK
    t_per_s = n_table // ns
    mesh = plsc.VectorSubcoreMesh(core_axis_name="c", subcore_axis_name="s")

    @functools.partial(
        pl.kernel,
        mesh=mesh,
        out_type=jax.ShapeDtypeStruct((n_rows, d), jnp.float32),
        scratch_types=[
            pltpu.VMEM((n_chunks, _CHUNK), jnp.int32),
            pltpu.VMEM((3, _CHUNK, d), jnp.float32),
            pltpu.VMEM_SHARED((n_table, d), jnp.float32),
            pltpu.SemaphoreType.DMA((3,)),
            pltpu.SemaphoreType.DMA((3,)),
        ],
    )
    def gather_kernel(table_hbm, idx_hbm, out_hbm, idx_v, rows_v, tab_sh,
                      gsem, wsem):
        sid = lax.axis_index("s")
        wid = sid * nc + lax.axis_index("c")
        base = wid * per_w
        pltpu.sync_copy(
            table_hbm.at[pl.ds(sid * t_per_s, t_per_s)],
            tab_sh.at[pl.ds(sid * t_per_s, t_per_s)],
        )
        pltpu.sync_copy(idx_hbm.at[wid], idx_v)
        plsc.subcore_barrier()

        def fire_gather(u, slot):
            pltpu.async_copy(
                tab_sh.at[idx_v.at[u]], rows_v.at[slot], gsem.at[slot]
            )

        def wait_gather(slot):
            pltpu.make_async_copy(
                out_hbm.at[pl.ds(0, _CHUNK)], rows_v.at[slot], gsem.at[slot]
            ).wait()

        def write_desc(u, slot):
            return pltpu.make_async_copy(
                rows_v.at[slot],
                out_hbm.at[pl.ds(base + u * _CHUNK, _CHUNK)],
                wsem.at[slot],
            )

        fire_gather(0, 0)
        fire_gather(1, 1)

        def body(u, _):
            slot = lax.rem(u, 3)

            @pl.when(u >= 1)
            def _():
                write_desc(u - 1, lax.rem(u - 1, 3)).wait()

            @pl.when(u + 2 < n_chunks)
            def _():
                fire_gather(u + 2, lax.rem(u + 2, 3))

            wait_gather(slot)
            write_desc(u, slot).start()
            return 0

        lax.fori_loop(0, n_chunks, body, 0)
        write_desc(n_chunks - 1, lax.rem(n_chunks - 1, 3)).wait()

    return gather_kernel


def kernel(pe, timesteps, index_select):
    if timesteps.ndim == 1:
        return pe[: timesteps.shape[0]]
    b, l = timesteps.shape
    n = b * l
    d = pe.shape[1]
    idx = (timesteps.reshape(-1) * index_select).astype(jnp.int32)
    info = plsc.get_sparse_core_info()
    nw = info.num_cores * info.num_subcores
    idx3d = idx.reshape(nw, n // (nw * _CHUNK), _CHUNK)
    out = _make_gather(n, d, pe.shape[0])(pe, idx3d)
    return out.reshape(b, l, d)
